# baseline (speedup 1.0000x reference)
import functools

import jax
import jax.numpy as jnp
from jax import lax
from jax.experimental import pallas as pl
from jax.experimental.pallas import tpu as pltpu
from jax.experimental.pallas import tpu_sc as plsc

D = 128
SEQ = 50
SEQ_PAD = 56
ENT_PER = 4
G_SPLIT = 2
NBUF = 4
AHEAD = 2
NW = 32
NREP = 16
TC_BLK = 64


@functools.partial(jax.jit, static_argnames=("ent_per_w",))
def _sc_gather(table, idx, ent_per_w):
    n_groups = ent_per_w // ENT_PER
    n_outer = n_groups // NBUF
    assert n_outer * NBUF == n_groups
    idx_per_w = ent_per_w * SEQ_PAD
    rows_per_slot = ENT_PER * SEQ_PAD
    rows_per_stream = rows_per_slot // G_SPLIT
    mesh = plsc.VectorSubcoreMesh(core_axis_name="c", subcore_axis_name="s")

    @functools.partial(
        pl.kernel,
        mesh=mesh,
        out_type=jax.ShapeDtypeStruct((NW * idx_per_w, D), jnp.float32),
        scratch_types=(
            [pltpu.VMEM((idx_per_w,), jnp.int32),
             pltpu.VMEM((NBUF, rows_per_slot, D), jnp.float32)]
            + [pltpu.SemaphoreType.DMA] * (2 * NBUF)
        ),
    )
    def k(table_hbm, idx_hbm, out_hbm, idx_v, rows_v, *sems):
        gsem = sems[:NBUF]
        osem = sems[NBUF:]
        wid = lax.axis_index("s") * 2 + lax.axis_index("c")
        row_base = wid * idx_per_w
        pltpu.sync_copy(idx_hbm.at[wid], idx_v)

        def gathers(s, slot):
            return [
                pltpu.make_async_copy(
                    table_hbm.at[idx_v.at[pl.ds(
                        s * rows_per_slot + j * rows_per_stream,
                        rows_per_stream)]],
                    rows_v.at[slot, pl.ds(j * rows_per_stream,
                                          rows_per_stream)],
                    gsem[slot])
                for j in range(G_SPLIT)
            ]

        def out_copy(slot, s):
            return pltpu.make_async_copy(
                rows_v.at[slot],
                out_hbm.at[pl.ds(row_base + s * rows_per_slot,
                                 rows_per_slot)],
                osem[slot])

        for h in range(AHEAD):
            for c in gathers(h, h):
                c.start()

        def body(t, carry):
            for b in range(NBUF):
                s = t * NBUF + b
                sh = (b + AHEAD) % NBUF
                h = s + AHEAD

                @pl.when(h < n_groups)
                def _():
                    @pl.when(h >= NBUF)
                    def _():
                        out_copy(sh, 0).wait()
                    for c in gathers(h, sh):
                        c.start()

                for c in gathers(s, b):
                    c.wait()
                out_copy(b, s).start()
            return carry

        lax.fori_loop(0, n_outer, body, 0)

        for b in range(NBUF):
            out_copy(b, 0).wait()

    return k(table, idx)


def _tc_slice(x_pad, bsz):
    def body(x_ref, o_ref):
        o_ref[...] = x_ref[:, :SEQ, :]

    return pl.pallas_call(
        body,
        grid=(bsz // TC_BLK,),
        in_specs=[pl.BlockSpec((TC_BLK, SEQ_PAD, D), lambda i: (i, 0, 0))],
        out_specs=pl.BlockSpec((TC_BLK, SEQ, D), lambda i: (i, 0, 0)),
        out_shape=jax.ShapeDtypeStruct((bsz, SEQ, D), jnp.float32),
    )(x_pad)


def kernel(indices, embed_weight):
    bsz, seq = indices.shape
    v = embed_weight.shape[0]
    ent_per_w = bsz // NW
    table_rep = jnp.tile(embed_weight, (NREP, 1))
    idx_p = jnp.pad(indices.astype(jnp.int32), ((0, 0), (0, SEQ_PAD - seq)))
    idx_w = idx_p.reshape(NW, ent_per_w * SEQ_PAD)
    rep_off = (jnp.arange(NW, dtype=jnp.int32) % NREP * v)[:, None]
    flat = _sc_gather(table_rep, idx_w + rep_off, ent_per_w)
    return _tc_slice(flat.reshape(bsz, SEQ_PAD, D), bsz)


# speedup vs baseline: 1.1601x; 1.1601x over previous
import functools

import jax
import jax.numpy as jnp
from jax import lax
from jax.experimental import pallas as pl
from jax.experimental.pallas import tpu as pltpu
from jax.experimental.pallas import tpu_sc as plsc

D = 128
SEQ = 50
SEQ_PAD = 56
CHUNK = 128
NBUF = 4
AHEAD = 2
NW = 32
NREP = 16


@functools.partial(jax.jit, static_argnames=("n_chunks",))
def _sc_gather(table, idx, n_chunks):
    rows_per_w = n_chunks * CHUNK
    n_outer = n_chunks // NBUF
    assert n_outer * NBUF == n_chunks
    mesh = plsc.VectorSubcoreMesh(core_axis_name="c", subcore_axis_name="s")

    @functools.partial(
        pl.kernel,
        mesh=mesh,
        out_type=jax.ShapeDtypeStruct((NW * rows_per_w, D), jnp.float32),
        scratch_types=(
            [pltpu.VMEM((n_chunks, CHUNK), jnp.int32),
             pltpu.VMEM((NBUF, CHUNK, D), jnp.float32)]
            + [pltpu.SemaphoreType.DMA] * (2 * NBUF)
        ),
    )
    def k(table_hbm, idx_hbm, out_hbm, idx_v, rows_v, *sems):
        gsem = sems[:NBUF]
        osem = sems[NBUF:]
        wid = lax.axis_index("s") * 2 + lax.axis_index("c")
        base = wid * rows_per_w
        pltpu.sync_copy(idx_hbm.at[wid], idx_v)

        def gather(g, slot):
            return pltpu.make_async_copy(
                table_hbm.at[idx_v.at[g]], rows_v.at[slot], gsem[slot])

        def out_copy(slot, g):
            return pltpu.make_async_copy(
                rows_v.at[slot],
                out_hbm.at[pl.ds(base + g * CHUNK, CHUNK)],
                osem[slot])

        for h in range(AHEAD):
            gather(h, h).start()

        def body(t, carry):
            for b in range(NBUF):
                g = t * NBUF + b
                sh = (b + AHEAD) % NBUF
                h = g + AHEAD

                @pl.when(h < n_chunks)
                def _():
                    @pl.when(h >= NBUF)
                    def _():
                        out_copy(sh, 0).wait()
                    gather(h, sh).start()

                gather(g, b).wait()
                out_copy(b, g).start()
            return carry

        lax.fori_loop(0, n_outer, body, 0)

        for b in range(NBUF):
            out_copy(b, 0).wait()

    return k(table, idx)


def kernel(indices, embed_weight):
    bsz, seq = indices.shape
    v = embed_weight.shape[0]
    table_rep = jnp.tile(embed_weight, (NREP, 1))
    idx_p = jnp.pad(indices.astype(jnp.int32), ((0, 0), (0, SEQ_PAD - seq)))
    n_chunks = bsz * SEQ_PAD // (NW * CHUNK)
    idx_w = idx_p.reshape(NW, n_chunks, CHUNK)
    rep_off = (jnp.arange(NW, dtype=jnp.int32) % NREP * v).reshape(NW, 1, 1)
    flat = _sc_gather(table_rep, idx_w + rep_off, n_chunks)
    return flat.reshape(bsz, SEQ_PAD, D)[:, :SEQ, :]
